# Initial kernel scaffold; baseline (speedup 1.0000x reference)
import jax
import jax.numpy as jnp
from jax.experimental import pallas as pl


def kernel(x, edge_index, edge_attr, Wl, bl, Wr, br, We, att, bias):
    raise NotImplementedError("write your pallas kernel here")


# speedup vs baseline: 1.2057x; 1.2057x over previous
import functools

import jax
import jax.numpy as jnp
from jax import lax
from jax.experimental import pallas as pl
from jax.experimental.pallas import tpu as pltpu
from jax.experimental.pallas import tpu_sc as plsc

N = 10000
E = 160000
F_IN = 256
D_E = 16
H = 4
C = 64
HC = H * C

NB = 64
CH_K2 = 79
PER_W = NB * CH_K2
EPAD = 32 * PER_W
NACC = 10080
NPW = NACC // 16
CH_K4 = 158

_f32 = jnp.float32
_i32 = jnp.int32


def _k1_node_body(x_ref, wl_ref, bl_ref, wr_ref, br_ref, xl_ref, xr_ref):
    x = x_ref[...]
    xl = jnp.dot(x, wl_ref[...], preferred_element_type=_f32) + bl_ref[...]
    xr = jnp.dot(x, wr_ref[...], preferred_element_type=_f32) + br_ref[...]
    xl_ref[0] = xl[:, :128]
    xl_ref[1] = xl[:, 128:]
    xr_ref[...] = xr


def _k1_node(x, Wl, bl, Wr, br):
    blk = 1000
    grid = N // blk
    return pl.pallas_call(
        _k1_node_body,
        grid=(grid,),
        in_specs=[
            pl.BlockSpec((blk, F_IN), lambda i: (i, 0)),
            pl.BlockSpec((F_IN, HC), lambda i: (0, 0)),
            pl.BlockSpec((HC,), lambda i: (0,)),
            pl.BlockSpec((F_IN, HC), lambda i: (0, 0)),
            pl.BlockSpec((HC,), lambda i: (0,)),
        ],
        out_specs=[
            pl.BlockSpec((2, blk, 128), lambda i: (0, i, 0)),
            pl.BlockSpec((blk, HC), lambda i: (i, 0)),
        ],
        out_shape=[
            jax.ShapeDtypeStruct((2, N, 128), _f32),
            jax.ShapeDtypeStruct((N, HC), _f32),
        ],
    )(x, Wl, bl, Wr, br)


def _k1_edge_body(ea_ref, we_ref, e_ref):
    e_ref[...] = jnp.dot(ea_ref[...], we_ref[...], preferred_element_type=_f32)


def _k1_edge(ea_pad, We):
    blk = 1024
    grid = EPAD // blk
    return pl.pallas_call(
        _k1_edge_body,
        grid=(grid,),
        in_specs=[
            pl.BlockSpec((blk, D_E), lambda i: (i, 0)),
            pl.BlockSpec((D_E, HC), lambda i: (0, 0)),
        ],
        out_specs=pl.BlockSpec((blk, HC), lambda i: (i, 0)),
        out_shape=jax.ShapeDtypeStruct((EPAD, HC), _f32),
    )(ea_pad, We)


def _k3_body(deg_ref, attr_ref, dpart_ref, xl_ref, xr_ref, we_ref, att_ref,
             denom_ref, wloop_ref, anl_ref):
    deg = deg_ref[0] + deg_ref[1]
    attr_sum = attr_ref[0] + attr_ref[1]
    loop_attr = attr_sum / jnp.maximum(deg, 1.0)
    e_loop = jnp.dot(loop_attr, we_ref[...], preferred_element_type=_f32)
    xl = jnp.concatenate([xl_ref[0], xl_ref[1]], axis=1)
    m = xl + xr_ref[...] + e_loop
    g = jnp.where(m > 0, m, 0.2 * m)
    att_flat = att_ref[...]
    sel = (lax.broadcasted_iota(_i32, (HC, H), 0) // C ==
           lax.broadcasted_iota(_i32, (HC, H), 1)).astype(_f32)
    logit = jnp.dot(g * att_flat, sel, preferred_element_type=_f32)
    w_loop = jnp.exp(jnp.minimum(logit, 60.0))
    denom = dpart_ref[0] + dpart_ref[1] + w_loop
    denom_ref[...] = denom
    wloop_ref[...] = w_loop
    anl_ref[...] = w_loop / denom


def _k3(deg_p, attr_p, denom_p, xl_ab, xr, We, att):
    blk = 1000
    grid = N // blk
    return pl.pallas_call(
        _k3_body,
        grid=(grid,),
        in_specs=[
            pl.BlockSpec((2, blk, 1), lambda i: (0, i, 0)),
            pl.BlockSpec((2, blk, D_E), lambda i: (0, i, 0)),
            pl.BlockSpec((2, blk, H), lambda i: (0, i, 0)),
            pl.BlockSpec((2, blk, 128), lambda i: (0, i, 0)),
            pl.BlockSpec((blk, HC), lambda i: (i, 0)),
            pl.BlockSpec((D_E, HC), lambda i: (0, 0)),
            pl.BlockSpec((1, HC), lambda i: (0, 0)),
        ],
        out_specs=[
            pl.BlockSpec((blk, H), lambda i: (i, 0)),
            pl.BlockSpec((blk, H), lambda i: (i, 0)),
            pl.BlockSpec((blk, H), lambda i: (i, 0)),
        ],
        out_shape=[
            jax.ShapeDtypeStruct((N, H), _f32),
            jax.ShapeDtypeStruct((N, H), _f32),
            jax.ShapeDtypeStruct((N, H), _f32),
        ],
    )(deg_p, attr_p, denom_p, xl_ab, xr, We, att)


def _k6_body(op_ref, xl_ref, wloop_ref, denom_ref, bias_ref, out_ref):
    sel = (lax.broadcasted_iota(_i32, (H, HC), 1) // C ==
           lax.broadcasted_iota(_i32, (H, HC), 0)).astype(_f32)
    w_rep = jnp.dot(wloop_ref[...], sel, preferred_element_type=_f32)
    rden = jnp.dot(1.0 / denom_ref[...], sel, preferred_element_type=_f32)
    un = jnp.concatenate([op_ref[0], op_ref[1]], axis=1)
    xl = jnp.concatenate([xl_ref[0], xl_ref[1]], axis=1)
    out = (un + w_rep * xl) * rden + bias_ref[...]
    out_ref[...] = jnp.maximum(out, 0.0)


def _k6(out_p, xl_ab, w_loop, denom, bias):
    blk = 1000
    grid = N // blk
    return pl.pallas_call(
        _k6_body,
        grid=(grid,),
        in_specs=[
            pl.BlockSpec((2, blk, 128), lambda i: (0, i, 0)),
            pl.BlockSpec((2, blk, 128), lambda i: (0, i, 0)),
            pl.BlockSpec((blk, H), lambda i: (i, 0)),
            pl.BlockSpec((blk, H), lambda i: (i, 0)),
            pl.BlockSpec((HC,), lambda i: (0,)),
        ],
        out_specs=pl.BlockSpec((blk, HC), lambda i: (i, 0)),
        out_shape=jax.ShapeDtypeStruct((N, HC), _f32),
    )(out_p, xl_ab, w_loop, denom, bias)


def _k2_scaffold(xl_ab, xr, e_pad, src_pad, dst_g, dst_acc, ea_pad, att):
    xl = jnp.concatenate([xl_ab[0], xl_ab[1]], axis=1)
    m = xl[src_pad] + xr[dst_g] + e_pad
    g = jnp.where(m > 0, m, 0.2 * m)
    logit = (g.reshape(EPAD, H, C) * att[None]).sum(-1)
    w = jnp.exp(jnp.minimum(logit, 60.0))
    deg_p = jax.ops.segment_sum(jnp.ones((EPAD,), _f32), dst_acc, num_segments=NACC)
    attr_p = jax.ops.segment_sum(ea_pad, dst_acc, num_segments=NACC)
    denom_p = jax.ops.segment_sum(w, dst_acc, num_segments=NACC)
    z = jnp.zeros_like
    return (w,
            jnp.stack([deg_p, z(deg_p)])[:, :, None],
            jnp.stack([attr_p, z(attr_p)]),
            jnp.stack([denom_p, z(denom_p)]))


def _k4_scaffold(xl_flat, src_pad, dst_acc, w, denom_pad):
    rows_a = xl_flat[src_pad]
    rows_b = xl_flat[src_pad + N]
    wa = jnp.repeat(w[:, 0:2], 64, axis=1)
    wb = jnp.repeat(w[:, 2:4], 64, axis=1)
    pa = jax.ops.segment_sum(rows_a * wa, dst_acc, num_segments=NACC)
    pb = jax.ops.segment_sum(rows_b * wb, dst_acc, num_segments=NACC)
    alpha = w / denom_pad[dst_acc]
    return jnp.stack([pa, pb]), alpha


def kernel(x, edge_index, edge_attr, Wl, bl, Wr, br, We, att, bias):
    src0 = edge_index[0]
    dst0 = edge_index[1]
    npad = EPAD - E
    pad_i = jnp.arange(npad, dtype=_i32)
    src_pad = jnp.concatenate([src0, pad_i % N])
    dst_g = jnp.concatenate([dst0, pad_i % N])
    dst_acc = jnp.concatenate([dst0, N + (pad_i % 64)])
    ea_pad = jnp.concatenate([edge_attr, jnp.zeros((npad, D_E), _f32)])

    xl_ab, xr = _k1_node(x, Wl, bl, Wr, br)
    e_pad = _k1_edge(ea_pad, We)

    w, deg_p, attr_p, denom_p = _k2_scaffold(xl_ab, xr, e_pad, src_pad, dst_g, dst_acc, ea_pad, att)

    denom, w_loop, alpha_n_loop = _k3(
        deg_p[:, :N], attr_p[:, :N], denom_p[:, :N], xl_ab, xr, We,
        att.reshape(1, HC))

    denom_pad = jnp.concatenate([denom, jnp.ones((NACC - N, H), _f32)])
    xl_flat = xl_ab.reshape(2 * N, 128)
    out_p, alpha_e = _k4_scaffold(xl_flat, src_pad, dst_acc, w, denom_pad)

    out = _k6(out_p[:, :N], xl_ab, w_loop, denom, bias)

    loop_idx = jnp.arange(N, dtype=src0.dtype)
    ei_out = jnp.stack([jnp.concatenate([src0, loop_idx]),
                        jnp.concatenate([dst0, loop_idx])])
    alpha = jnp.concatenate([alpha_e[:E], alpha_n_loop], axis=0)
    return (out, (ei_out, alpha))


# speedup vs baseline: 2.8668x; 2.3778x over previous
import functools

import jax
import jax.numpy as jnp
from jax import lax
from jax.experimental import pallas as pl
from jax.experimental.pallas import tpu as pltpu
from jax.experimental.pallas import tpu_sc as plsc

N = 10000
E = 160000
F_IN = 256
D_E = 16
H = 4
C = 64
HC = H * C

NB = 64
CH_K2 = 79
PER_W = NB * CH_K2
EPAD = 32 * PER_W
NACC = 10112
NPW = NACC // 16
CH_K4 = 158

_f32 = jnp.float32
_i32 = jnp.int32


def _k1_node_body(x_ref, wl_ref, bl_ref, wr_ref, br_ref,
                  xl_ref, xlf_ref, xr_ref):
    x = x_ref[...]
    xl = jnp.dot(x, wl_ref[...], preferred_element_type=_f32) + bl_ref[...]
    xr = jnp.dot(x, wr_ref[...], preferred_element_type=_f32) + br_ref[...]
    xl_ref[0] = xl[:, :128]
    xl_ref[1] = xl[:, 128:]
    xlf_ref[...] = xl
    xr_ref[...] = xr


def _k1_node(x, Wl, bl, Wr, br):
    blk = 1000
    grid = N // blk
    return pl.pallas_call(
        _k1_node_body,
        grid=(grid,),
        in_specs=[
            pl.BlockSpec((blk, F_IN), lambda i: (i, 0)),
            pl.BlockSpec((F_IN, HC), lambda i: (0, 0)),
            pl.BlockSpec((HC,), lambda i: (0,)),
            pl.BlockSpec((F_IN, HC), lambda i: (0, 0)),
            pl.BlockSpec((HC,), lambda i: (0,)),
        ],
        out_specs=[
            pl.BlockSpec((2, blk, 128), lambda i: (0, i, 0)),
            pl.BlockSpec((blk, HC), lambda i: (i, 0)),
            pl.BlockSpec((blk, HC), lambda i: (i, 0)),
        ],
        out_shape=[
            jax.ShapeDtypeStruct((2, N, 128), _f32),
            jax.ShapeDtypeStruct((N, HC), _f32),
            jax.ShapeDtypeStruct((N, HC), _f32),
        ],
    )(x, Wl, bl, Wr, br)


def _k1_edge_body(ea_ref, we_ref, e_ref):
    e_ref[...] = jnp.dot(ea_ref[...], we_ref[...], preferred_element_type=_f32)


def _k1_edge(ea_pad, We):
    blk = 1024
    grid = EPAD // blk
    return pl.pallas_call(
        _k1_edge_body,
        grid=(grid,),
        in_specs=[
            pl.BlockSpec((blk, D_E), lambda i: (i, 0)),
            pl.BlockSpec((D_E, HC), lambda i: (0, 0)),
        ],
        out_specs=pl.BlockSpec((blk, HC), lambda i: (i, 0)),
        out_shape=jax.ShapeDtypeStruct((EPAD, HC), _f32),
    )(ea_pad, We)


def _k3_body(deg_ref, attr_ref, dpart_ref, xl_ref, xr_ref, we_ref, att_ref,
             denom_ref, wloop_ref, anl_ref):
    deg = deg_ref[0] + deg_ref[1]
    attr_sum = attr_ref[0] + attr_ref[1]
    loop_attr = attr_sum / jnp.maximum(deg, 1.0)
    e_loop = jnp.dot(loop_attr, we_ref[...], preferred_element_type=_f32)
    xl = jnp.concatenate([xl_ref[0], xl_ref[1]], axis=1)
    m = xl + xr_ref[...] + e_loop
    g = jnp.where(m > 0, m, 0.2 * m)
    att_flat = att_ref[...]
    sel = (lax.broadcasted_iota(_i32, (HC, H), 0) // C ==
           lax.broadcasted_iota(_i32, (HC, H), 1)).astype(_f32)
    logit = jnp.dot(g * att_flat, sel, preferred_element_type=_f32)
    w_loop = jnp.exp(jnp.minimum(logit, 60.0))
    denom = dpart_ref[0] + dpart_ref[1] + w_loop
    denom_ref[...] = denom
    wloop_ref[...] = w_loop
    anl_ref[...] = w_loop / denom


def _k3(deg_p, attr_p, denom_p, xl_ab, xr, We, att):
    blk = 1000
    grid = N // blk
    return pl.pallas_call(
        _k3_body,
        grid=(grid,),
        in_specs=[
            pl.BlockSpec((2, blk, 1), lambda i: (0, i, 0)),
            pl.BlockSpec((2, blk, D_E), lambda i: (0, i, 0)),
            pl.BlockSpec((2, blk, H), lambda i: (0, i, 0)),
            pl.BlockSpec((2, blk, 128), lambda i: (0, i, 0)),
            pl.BlockSpec((blk, HC), lambda i: (i, 0)),
            pl.BlockSpec((D_E, HC), lambda i: (0, 0)),
            pl.BlockSpec((1, HC), lambda i: (0, 0)),
        ],
        out_specs=[
            pl.BlockSpec((blk, H), lambda i: (i, 0)),
            pl.BlockSpec((blk, H), lambda i: (i, 0)),
            pl.BlockSpec((blk, H), lambda i: (i, 0)),
        ],
        out_shape=[
            jax.ShapeDtypeStruct((N, H), _f32),
            jax.ShapeDtypeStruct((N, H), _f32),
            jax.ShapeDtypeStruct((N, H), _f32),
        ],
    )(deg_p, attr_p, denom_p, xl_ab, xr, We, att)


def _k6_body(op_ref, xl_ref, wloop_ref, denom_ref, bias_ref, out_ref):
    sel = (lax.broadcasted_iota(_i32, (H, HC), 1) // C ==
           lax.broadcasted_iota(_i32, (H, HC), 0)).astype(_f32)
    w_rep = jnp.dot(wloop_ref[...], sel, preferred_element_type=_f32)
    rden = jnp.dot(1.0 / denom_ref[...], sel, preferred_element_type=_f32)
    un = jnp.concatenate([op_ref[0], op_ref[1]], axis=1)
    xl = jnp.concatenate([xl_ref[0], xl_ref[1]], axis=1)
    out = (un + w_rep * xl) * rden + bias_ref[...]
    out_ref[...] = jnp.maximum(out, 0.0)


def _k6(out_p, xl_ab, w_loop, denom, bias):
    blk = 1000
    grid = N // blk
    return pl.pallas_call(
        _k6_body,
        grid=(grid,),
        in_specs=[
            pl.BlockSpec((2, blk, 128), lambda i: (0, i, 0)),
            pl.BlockSpec((2, blk, 128), lambda i: (0, i, 0)),
            pl.BlockSpec((blk, H), lambda i: (i, 0)),
            pl.BlockSpec((blk, H), lambda i: (i, 0)),
            pl.BlockSpec((HC,), lambda i: (0,)),
        ],
        out_specs=pl.BlockSpec((blk, HC), lambda i: (i, 0)),
        out_shape=jax.ShapeDtypeStruct((N, HC), _f32),
    )(out_p, xl_ab, w_loop, denom, bias)


_MESH = plsc.VectorSubcoreMesh(core_axis_name="c", subcore_axis_name="s")


def _k2_body(xlh, xrh, eh, srch, dgh, dah, eah, atth,
             w_out, comb_out, attr_out,
             srcv, dgv, dav, eiA, eiB, degv, xlab, xrb, eb, eab,
             atw0, atw1, atw2, atw3, atw4, atw5, atw6, atw7,
             ati0, ati1, ati2, ati3, ati4, ati5, ati6, ati7,
             wrA, wrB, ones_b, attv,
             comb_acc, attr_acc, zb1, sem):
    wrh = (wrA, wrB)
    atw = (atw0, atw1, atw2, atw3, atw4, atw5, atw6, atw7)
    ati = (ati0, ati1, ati2, ati3, ati4, ati5, ati6, ati7)
    c = lax.axis_index("c")
    s = lax.axis_index("s")
    wid = s * 2 + c
    iota16 = lax.iota(_i32, 16)
    lmask = [iota16 == i for i in range(16)]
    zeros16 = jnp.zeros((16,), _f32)
    ones16 = jnp.ones((16,), _f32)

    def _zrow(i, carry):
        zb1[pl.ds(i * 16, 16)] = zeros16
        return carry
    lax.fori_loop(0, NPW, _zrow, 0)
    sl_acc1 = pl.ds(s * NPW * 16, NPW * 16)
    pltpu.sync_copy(zb1, comb_acc.at[sl_acc1])
    pltpu.sync_copy(zb1, attr_acc.at[sl_acc1])

    pltpu.sync_copy(atth, attv)
    att_vecs = [attv[pl.ds(q * 16, 16)] for q in range(16)]
    for g in range(NB // 16):
        ones_b[pl.ds(g * 16, 16)] = ones16
    plsc.subcore_barrier()

    @pl.loop(0, CH_K2, unroll=1)
    def chunk(ci):
        base = wid * PER_W + ci * NB
        d0 = pltpu.async_copy(srch.at[pl.ds(base, NB)], srcv, sem)
        d0b = pltpu.async_copy(dgh.at[pl.ds(base, NB)], dgv, sem)
        d0c = pltpu.async_copy(dah.at[pl.ds(base, NB)], dav, sem)
        d4 = pltpu.async_copy(eh.at[pl.ds(base, NB)], eb, sem)
        d5 = pltpu.async_copy(eah.at[pl.ds(base, NB)], eab, sem)
        d0.wait(); d0b.wait(); d0c.wait()
        for g in range(NB // 16):
            sl = pl.ds(g * 16, 16)
            srcv[sl] = srcv[sl] + 0
            dgv[sl] = dgv[sl] + 0
            dv16 = dav[sl]
            degv[sl] = dv16 * 16 + 4
            eit = eiA if g < 2 else eiB
            for sub in range(4):
                rep = dv16.at[(iota16 >> 2) + sub * 4].get(
                    mode="promise_in_bounds")
                eit[pl.ds(((g % 2) * 4 + sub) * 16, 16)] = (
                    rep * 16 + (iota16 & 3))
        d1 = pltpu.async_copy(xlh.at[srcv], xlab, sem)
        d3 = pltpu.async_copy(xrh.at[dgv], xrb, sem)
        d4.wait(); d5.wait()
        d1.wait(); d3.wait()
        for g in range(NB // 16):
            dvg = dav[pl.ds(g * 16, 16)]
            for k in range(16):
                j = g * 16 + k
                rep = dvg.at[jnp.full((16,), k, _i32)].get(
                    mode="promise_in_bounds")
                b, slot = j // 8, j % 8
                atw[b][pl.ds(slot * 16, 16)] = eab[j, pl.ds(0, 16)]
                ati[b][pl.ds(slot * 16, 16)] = rep * 16 + iota16

        for half in range(2):
            def ggbody(gg, carry2, _half=half):
                gb = _half * 8 + gg
                lv = zeros16
                for k in range(4):
                    j = gb * 4 + k
                    for h in range(H):
                        acc = zeros16
                        for q in range(4):
                            off = h * 64 + q * 16
                            sl = pl.ds(off, 16)
                            m = xlab[j, sl] + xrb[j, sl] + eb[j, sl]
                            gl = jnp.maximum(m, 0.2 * m)
                            acc = acc + gl * att_vecs[h * 4 + q]
                        for st in (8, 4, 2, 1):
                            acc = acc + acc.at[(iota16 + st) & 15].get(mode="promise_in_bounds")
                        lv = jnp.where(lmask[k * 4 + h], acc, lv)
                w16 = jnp.exp(jnp.minimum(lv, 60.0))
                wrh[_half][pl.ds(gg * 16, 16)] = w16
                return carry2
            lax.fori_loop(0, 8, ggbody, 0)

        pltpu.sync_copy(wrA, w_out.at[pl.ds(base * 4, 128)])
        pltpu.sync_copy(wrB, w_out.at[pl.ds(base * 4 + 128, 128)])
        pltpu.sync_copy(wrA, comb_acc.at[eiA], add=True)
        pltpu.sync_copy(wrB, comb_acc.at[eiB], add=True)
        pltpu.sync_copy(ones_b, comb_acc.at[degv], add=True)
        for b in range(8):
            pltpu.sync_copy(atw[b], attr_acc.at[ati[b]], add=True)

    plsc.subcore_barrier()
    pltpu.sync_copy(comb_acc.at[sl_acc1], zb1)
    pltpu.sync_copy(zb1, comb_out.at[c, sl_acc1])
    pltpu.sync_copy(attr_acc.at[sl_acc1], zb1)
    pltpu.sync_copy(zb1, attr_out.at[c, sl_acc1])


def _k2_sc(xl_full, xr, e_pad, src_pad, dst_g, dst_acc, ea128, att_flat):
    f = pl.kernel(
        _k2_body,
        out_type=[
            jax.ShapeDtypeStruct((EPAD * 4,), _f32),
            jax.ShapeDtypeStruct((2, NACC * 16), _f32),
            jax.ShapeDtypeStruct((2, NACC * 16), _f32),
        ],
        mesh=_MESH,
        scratch_types=[
            pltpu.VMEM((NB,), _i32),
            pltpu.VMEM((NB,), _i32),
            pltpu.VMEM((NB,), _i32),
            pltpu.VMEM((128,), _i32),
            pltpu.VMEM((128,), _i32),
            pltpu.VMEM((NB,), _i32),
            pltpu.VMEM((NB, HC), _f32),
            pltpu.VMEM((NB, HC), _f32),
            pltpu.VMEM((NB, HC), _f32),
            pltpu.VMEM((NB, 128), _f32),
        ] + [pltpu.VMEM((128,), _f32) for _ in range(8)]
          + [pltpu.VMEM((128,), _i32) for _ in range(8)] + [
            pltpu.VMEM((128,), _f32),
            pltpu.VMEM((128,), _f32),
            pltpu.VMEM((NB,), _f32),
            pltpu.VMEM((HC,), _f32),
            pltpu.VMEM_SHARED((NACC * 16,), _f32),
            pltpu.VMEM_SHARED((NACC * 16,), _f32),
            pltpu.VMEM((NPW * 16,), _f32),
            pltpu.SemaphoreType.DMA,
        ],
    )
    return f(xl_full, xr, e_pad, src_pad, dst_g, dst_acc, ea128, att_flat)


def _k4_body(xlf, srch, dah, wh, denh,
             out_hbm, alpha_out,
             srcv, dav, davs, rows, wrb, dden, abA, abB, acc, zb, sem):
    abh = (abA, abB)
    c = lax.axis_index("c")
    s = lax.axis_index("s")
    iota16 = lax.iota(_i32, 16)
    lmask = [iota16 == i for i in range(16)]
    zeros16 = jnp.zeros((16,), _f32)
    is0 = c == 0

    def _zrow(i, carry):
        for k in range(8):
            zb[i, pl.ds(k * 16, 16)] = zeros16
        return carry
    lax.fori_loop(0, 104, _zrow, 0)
    for t in range(6):
        pltpu.sync_copy(zb, acc.at[pl.ds(s * NPW + t * 104, 104)])
    pltpu.sync_copy(zb.at[pl.ds(0, 8)], acc.at[pl.ds(s * NPW + 624, 8)])
    plsc.subcore_barrier()

    @pl.loop(0, CH_K4, unroll=1)
    def chunk(ci):
        base = s * (CH_K4 * NB) + ci * NB
        d0 = pltpu.async_copy(srch.at[pl.ds(base, NB)], srcv, sem)
        d0b = pltpu.async_copy(dah.at[pl.ds(base, NB)], dav, sem)
        d0c = pltpu.async_copy(wh.at[pl.ds(base * 4, NB * 4)], wrb, sem)
        d0.wait(); d0b.wait(); d0c.wait()
        for g in range(NB // 16):
            sl = pl.ds(g * 16, 16)
            srcv[sl] = srcv[sl] + c * N
            davs[sl] = dav[sl] + 0
        d1 = pltpu.async_copy(xlf.at[srcv], rows, sem)

        @pl.when(is0)
        def _():
            pltpu.sync_copy(denh.at[davs], dden)
        d1.wait()

        c2 = 2 * c
        for half in range(2):
            def ggbody(gg, carry2, _half=half):
                gb = _half * 8 + gg
                wv16 = wrb[pl.ds(gb * 16, 16)]
                for k in range(4):
                    j = gb * 4 + k
                    base_i = jnp.full((16,), k * 4, _i32) + c2
                    w0s = wv16.at[base_i].get(mode="promise_in_bounds")
                    w1s = wv16.at[base_i + 1].get(mode="promise_in_bounds")
                    for fb in range(8):
                        ws = w0s if fb < 4 else w1s
                        sl = pl.ds(fb * 16, 16)
                        rows[j, sl] = rows[j, sl] * ws

                @pl.when(is0)
                def _():
                    den16 = zeros16
                    for k in range(4):
                        j = gb * 4 + k
                        dr = dden[j, pl.ds(0, 16)]
                        for h in range(H):
                            dsplat = dr.at[jnp.full((16,), h, _i32)].get(mode="promise_in_bounds")
                            den16 = jnp.where(lmask[k * 4 + h], dsplat, den16)
                    abh[_half][pl.ds(gg * 16, 16)] = wv16 / den16
                return carry2
            lax.fori_loop(0, 8, ggbody, 0)

        pltpu.sync_copy(rows, acc.at[davs], add=True)

        @pl.when(is0)
        def _():
            pltpu.sync_copy(abA, alpha_out.at[pl.ds(base * 4, 128)])
            pltpu.sync_copy(abB, alpha_out.at[pl.ds(base * 4 + 128, 128)])

    plsc.subcore_barrier()
    for t in range(6):
        slx = pl.ds(s * NPW + t * 104, 104)
        pltpu.sync_copy(acc.at[slx], zb)
        pltpu.sync_copy(zb, out_hbm.at[c, slx])
    slx = pl.ds(s * NPW + 624, 8)
    pltpu.sync_copy(acc.at[slx], zb.at[pl.ds(0, 8)])
    pltpu.sync_copy(zb.at[pl.ds(0, 8)], out_hbm.at[c, slx])


def _k4_sc(xl_flat, src_pad, dst_acc, w_flat, denp16):
    f = pl.kernel(
        _k4_body,
        out_type=[
            jax.ShapeDtypeStruct((2, NACC, 128), _f32),
            jax.ShapeDtypeStruct((EPAD * 4,), _f32),
        ],
        mesh=_MESH,
        scratch_types=[
            pltpu.VMEM((NB,), _i32),
            pltpu.VMEM((NB,), _i32),
            pltpu.VMEM((NB,), _i32),
            pltpu.VMEM((NB, 128), _f32),
            pltpu.VMEM((NB * 4,), _f32),
            pltpu.VMEM((NB, 128), _f32),
            pltpu.VMEM((128,), _f32),
            pltpu.VMEM((128,), _f32),
            pltpu.VMEM_SHARED((NACC, 128), _f32),
            pltpu.VMEM((104, 128), _f32),
            pltpu.SemaphoreType.DMA,
        ],
    )
    return f(xl_flat, src_pad, dst_acc, w_flat, denp16)

def kernel(x, edge_index, edge_attr, Wl, bl, Wr, br, We, att, bias):
    src0 = edge_index[0]
    dst0 = edge_index[1]
    npad = EPAD - E
    pad_i = jnp.arange(npad, dtype=_i32)
    src_pad = jnp.concatenate([src0, pad_i % N])
    dst_g = jnp.concatenate([dst0, pad_i % N])
    dst_acc = jnp.concatenate([dst0, N + (pad_i % 64)])
    ea_pad = jnp.concatenate([edge_attr, jnp.zeros((npad, D_E), _f32)])

    xl_ab, xl_full, xr = _k1_node(x, Wl, bl, Wr, br)
    e_pad = _k1_edge(ea_pad, We)
    xl_flat = xl_ab.reshape(2 * N, 128)

    ea128 = jnp.pad(ea_pad, ((0, 0), (0, 128 - D_E)))
    w_flat, comb, attr_flat = _k2_sc(
        xl_full, xr, e_pad, src_pad, dst_g, dst_acc, ea128, att.reshape(HC))
    attr_comb = attr_flat.reshape(2, NACC, 16)
    comb = comb.reshape(2, NACC, 16)
    deg_p = comb[:, :, 4:5]
    denom_p = comb[:, :, :4]

    denom, w_loop, alpha_n_loop = _k3(
        deg_p[:, :N], attr_comb[:, :N], denom_p[:, :N], xl_ab, xr, We,
        att.reshape(1, HC))

    denom_pad = jnp.concatenate([denom, jnp.ones((NACC - N, H), _f32)])
    denp16 = jnp.pad(denom_pad, ((0, 0), (0, 124)), constant_values=1.0)
    out_p, alpha_flat = _k4_sc(xl_flat, src_pad, dst_acc, w_flat, denp16)
    alpha_e = alpha_flat.reshape(EPAD, H)

    out = _k6(out_p[:, :N], xl_ab, w_loop, denom, bias)

    loop_idx = jnp.arange(N, dtype=src0.dtype)
    ei_out = jnp.stack([jnp.concatenate([src0, loop_idx]),
                        jnp.concatenate([dst0, loop_idx])])
    alpha = jnp.concatenate([alpha_e[:E], alpha_n_loop], axis=0)
    return (out, (ei_out, alpha))
